# baseline (speedup 1.0000x reference)
import functools

import jax
import jax.numpy as jnp
from jax import lax
from jax.experimental import pallas as pl
from jax.experimental.pallas import tpu as pltpu
from jax.experimental.pallas import tpu_sc as plsc

N = 10000
D = 128
E = 320000
L = 3

NC = 2
NS = 16
NW = NC * NS
EPW = E // NW
CHUNK = 50
NCHUNK = EPW // CHUNK
NBUF = 5
GCH = 20
NGROUP = NCHUNK // GCH
BPG = GCH // NBUF
RPT = 624
TAIL = N - NS * RPT


def _sc_aggregate(h, ei, zeros):
    mesh = plsc.VectorSubcoreMesh(core_axis_name="c", subcore_axis_name="s")

    @functools.partial(
        pl.kernel,
        out_type=jax.ShapeDtypeStruct((NC, N, D), jnp.float32),
        mesh=mesh,
        scratch_types=[
            pltpu.VMEM_SHARED((N, D), jnp.float32),
            pltpu.VMEM((2 * GCH, CHUNK), jnp.int32),
        ] + [pltpu.VMEM((CHUNK, D), jnp.float32) for _ in range(NBUF)]
          + [pltpu.SemaphoreType.DMA for _ in range(2 * NBUF)],
    )
    def agg(h_hbm, e_hbm, z_hbm, out_hbm, aggr_sh, idx, *bufs_and_sems):
        rows = bufs_and_sems[:NBUF]
        gsem = bufs_and_sems[NBUF:2 * NBUF]
        ssem = bufs_and_sems[2 * NBUF:]
        c = lax.axis_index("c")
        s = lax.axis_index("s")
        wid = c * NS + s
        pltpu.sync_copy(z_hbm.at[pl.ds(s * RPT, RPT)],
                        aggr_sh.at[pl.ds(s * RPT, RPT)])

        @pl.when(s == NS - 1)
        def _():
            pltpu.sync_copy(z_hbm.at[pl.ds(NS * RPT, TAIL)],
                            aggr_sh.at[pl.ds(NS * RPT, TAIL)])
        plsc.subcore_barrier()

        def drain_scatters(b):
            pltpu.make_async_copy(rows[b], aggr_sh.at[idx.at[1]],
                                  ssem[b]).wait()

        def group(g, carry):
            @pl.when(g > 0)
            def _():
                for b in range(NBUF):
                    drain_scatters(b)

            base = wid * 2 * NCHUNK + g * 2 * GCH
            pltpu.sync_copy(e_hbm.at[pl.ds(base, 2 * GCH)], idx)

            def batch(t, carry2):
                for b in range(NBUF):
                    m = t * NBUF + b

                    @pl.when(t > 0)
                    def _():
                        drain_scatters(b)

                    pltpu.async_copy(h_hbm.at[idx.at[2 * m]], rows[b],
                                     gsem[b])
                for b in range(NBUF):
                    m = t * NBUF + b
                    pltpu.make_async_copy(h_hbm.at[idx.at[2 * m]], rows[b],
                                          gsem[b]).wait()
                    pltpu.async_copy(rows[b], aggr_sh.at[idx.at[2 * m + 1]],
                                     ssem[b], add=True)
                return carry2

            lax.fori_loop(0, BPG, batch, 0)
            return carry

        lax.fori_loop(0, NGROUP, group, 0)
        for b in range(NBUF):
            drain_scatters(b)
        plsc.subcore_barrier()
        pltpu.sync_copy(aggr_sh.at[pl.ds(s * RPT, RPT)],
                        out_hbm.at[c, pl.ds(s * RPT, RPT)])

        @pl.when(s == NS - 1)
        def _():
            pltpu.sync_copy(aggr_sh.at[pl.ds(NS * RPT, TAIL)],
                            out_hbm.at[c, pl.ds(NS * RPT, TAIL)])

    return agg(h, ei, zeros)


def _mlp_body(relu_out, h_ref, p_ref, w1_ref, b1_ref, g1_ref, be1_ref,
              w2_ref, b2_ref, g2_ref, be2_ref, o_ref):
    out = h_ref[...] + p_ref[0] + p_ref[1]
    z = jnp.dot(out, w1_ref[...], preferred_element_type=jnp.float32)
    z = z + b1_ref[...]
    m = jnp.mean(z, axis=0, keepdims=True)
    v = jnp.mean((z - m) ** 2, axis=0, keepdims=True)
    z = (z - m) * lax.rsqrt(v + 1e-5) * g1_ref[...] + be1_ref[...]
    z = jnp.maximum(z, 0.0)
    z = jnp.dot(z, w2_ref[...], preferred_element_type=jnp.float32)
    z = z + b2_ref[...]
    m2 = jnp.mean(z, axis=0, keepdims=True)
    v2 = jnp.mean((z - m2) ** 2, axis=0, keepdims=True)
    z = (z - m2) * lax.rsqrt(v2 + 1e-5) * g2_ref[...] + be2_ref[...]
    if relu_out:
        z = jnp.maximum(z, 0.0)
    o_ref[...] = z


def _mlp(h, parts, w1, b1, g1, be1, w2, b2, g2, be2, relu_out):
    return pl.pallas_call(
        functools.partial(_mlp_body, relu_out),
        out_shape=jax.ShapeDtypeStruct((N, D), jnp.float32),
    )(h, parts, w1, b1.reshape(1, D), g1.reshape(1, D), be1.reshape(1, D),
      w2, b2.reshape(1, D), g2.reshape(1, D), be2.reshape(1, D))


def kernel(x, edge_index, W1, b1, g1, be1, W2, b2, g2, be2):
    ei = edge_index.astype(jnp.int32).reshape(2, NW, NCHUNK, CHUNK)
    ei = jnp.transpose(ei, (1, 2, 0, 3)).reshape(NW * NCHUNK * 2, CHUNK)
    zeros = jnp.zeros((N, D), jnp.float32)
    h = x
    for l in range(L):
        parts = _sc_aggregate(h, ei, zeros)
        h = _mlp(h, parts, W1[l], b1[l], g1[l], be1[l],
                 W2[l], b2[l], g2[l], be2[l], l < L - 1)
    return h
